# Initial kernel scaffold; baseline (speedup 1.0000x reference)
import jax
import jax.numpy as jnp
from jax.experimental import pallas as pl


def kernel(inputs, targets, mask):
    raise NotImplementedError("write your pallas kernel here")


# speedup vs baseline: 6.3858x; 6.3858x over previous
import functools

import jax
import jax.numpy as jnp
from jax import lax
from jax.experimental import pallas as pl
from jax.experimental.pallas import tpu as pltpu
from jax.experimental.pallas import tpu_sc as plsc

NC = 2
NS = 16
NW = NC * NS
L = 16

N = 128 * 32768
E = N // NW
C = 8192
NBINS = 4096


def _histo_body(is_pass2, a_hbm, b_hbm, m_hbm, bsel_hbm, out_hbm,
                av, bv, mv, bselv, hc, hs):
    wid = lax.axis_index("s") * NC + lax.axis_index("c")
    base = wid * E
    zeros = jnp.zeros((L,), jnp.float32)
    ones = jnp.ones((L,), jnp.float32)

    def zero_body(j, _):
        hc[pl.ds(j * L, L)] = zeros
        hs[pl.ds(j * L, L)] = zeros
        return _
    lax.fori_loop(0, NBINS // L, zero_body, None)

    if is_pass2:
        pltpu.sync_copy(bsel_hbm, bselv)
        bsel = bselv[...]

    def chunk_body(c, _):
        off = base + c * C
        pltpu.sync_copy(a_hbm.at[pl.ds(off, C)], av)
        pltpu.sync_copy(b_hbm.at[pl.ds(off, C)], bv)
        pltpu.sync_copy(m_hbm.at[pl.ds(off, C)], mv)

        def inner(i, _):
            s = i * L
            a = av[pl.ds(s, L)]
            b = bv[pl.ds(s, L)]
            m = mv[pl.ds(s, L)]
            d = jnp.where(m != 0, jnp.abs(a - b), jnp.float32(-1.0))
            u = lax.bitcast_convert_type(d, jnp.int32)
            idx1 = lax.shift_right_logical(u, 20)
            if is_pass2:
                idx2 = lax.bitwise_and(lax.shift_right_logical(u, 9),
                                       jnp.int32(2047))
                idx = jnp.where(idx1 == bsel, idx2, jnp.int32(3000))
            else:
                idx = idx1
            plsc.addupdate_scatter(hc, [idx], ones)
            plsc.addupdate_scatter(hs, [idx], d)
            return _
        lax.fori_loop(0, C // L, inner, None)
        return _
    lax.fori_loop(0, E // C, chunk_body, None)

    pltpu.sync_copy(hc, out_hbm.at[wid, 0])
    pltpu.sync_copy(hs, out_hbm.at[wid, 1])


def _make_pass(is_pass2):
    mesh = plsc.VectorSubcoreMesh(core_axis_name="c", subcore_axis_name="s",
                                  num_cores=NC, num_subcores=NS)
    return pl.kernel(
        functools.partial(_histo_body, is_pass2),
        out_type=jax.ShapeDtypeStruct((NW, 2, NBINS), jnp.float32),
        mesh=mesh,
        scratch_types=[
            pltpu.VMEM((C,), jnp.float32),
            pltpu.VMEM((C,), jnp.float32),
            pltpu.VMEM((C,), jnp.int32),
            pltpu.VMEM((L,), jnp.int32),
            pltpu.VMEM((NBINS,), jnp.float32),
            pltpu.VMEM((NBINS,), jnp.float32),
        ],
        compiler_params=pltpu.CompilerParams(needs_layout_passes=False),
        name="ohem_histo2" if is_pass2 else "ohem_histo1",
    )


def _select_bin(cnt, k):
    cum = jnp.cumsum(cnt[::-1])[::-1]
    b = jnp.clip(jnp.sum((cum >= k).astype(jnp.int32)) - 1, 0, 2047)
    cump = jnp.concatenate([cum, jnp.zeros((1,), cum.dtype)])
    return b, cump[b + 1]


def kernel(inputs, targets, mask):
    a = inputs.reshape(-1)
    b = targets.reshape(-1)
    m = mask.reshape(-1).astype(jnp.int32)

    pass1 = _make_pass(False)
    pass2 = _make_pass(True)

    h1 = pass1(a, b, m, jnp.zeros((L,), jnp.int32))
    cnt1f = jnp.sum(h1[:, 0, :2048], axis=0)
    sum1 = jnp.sum(h1[:, 1, :2048], axis=0)
    cnt1 = cnt1f.astype(jnp.int32)
    num_selected = jnp.sum(cnt1)
    k = (num_selected * 6) // 10

    b1, cnt_gt1 = _select_bin(cnt1, k)
    sum_gt1 = jnp.sum(jnp.where(jnp.arange(2048) > b1, sum1, 0.0))

    h2 = pass2(a, b, m, jnp.full((L,), b1, jnp.int32))
    cnt2 = jnp.sum(h2[:, 0, :2048], axis=0).astype(jnp.int32)
    sum2 = jnp.sum(h2[:, 1, :2048], axis=0)

    b2, cnt_gt2 = _select_bin(cnt2, k - cnt_gt1)
    sum_gt2 = jnp.sum(jnp.where(jnp.arange(2048) > b2, sum2, 0.0))

    t_bits = (b1 << 20) | (b2 << 9) | 255
    t = lax.bitcast_convert_type(t_bits, jnp.float32)
    rem = (k - cnt_gt1 - cnt_gt2).astype(jnp.float32)
    total = sum_gt1 + sum_gt2 + rem * t
    return total / k.astype(jnp.float32)


# speedup vs baseline: 10.5036x; 1.6448x over previous
import functools

import jax
import jax.numpy as jnp
from jax import lax
from jax.experimental import pallas as pl
from jax.experimental.pallas import tpu as pltpu
from jax.experimental.pallas import tpu_sc as plsc

NC = 2
NS = 16
NW = NC * NS
L = 16

N = 128 * 32768
E = N // NW
C = 8192
NCH = E // C
NBINS = 4096
UNROLL = 8


def _histo_body(is_pass2, a_hbm, b_hbm, m_hbm, bsel_hbm, out_hbm,
                av0, av1, bv0, bv1, mv0, mv1, bselv, accv, hc, sa, sb, sm):
    abufs, bbufs, mbufs = (av0, av1), (bv0, bv1), (mv0, mv1)
    wid = lax.axis_index("s") * NC + lax.axis_index("c")
    base = wid * E
    zeros = jnp.zeros((L,), jnp.float32)
    ones = jnp.ones((L,), jnp.float32)

    def zero_body(j, _):
        hc[pl.ds(j * L, L)] = zeros
        return _
    lax.fori_loop(0, NBINS // L, zero_body, None, unroll=8)

    if is_pass2:
        pltpu.sync_copy(bsel_hbm, bselv)
        bsel = bselv[...]

    def start(c, slot):
        off = base + c * C
        return (pltpu.async_copy(a_hbm.at[pl.ds(off, C)], abufs[slot], sa.at[slot]),
                pltpu.async_copy(b_hbm.at[pl.ds(off, C)], bbufs[slot], sb.at[slot]),
                pltpu.async_copy(m_hbm.at[pl.ds(off, C)], mbufs[slot], sm.at[slot]))

    acc = zeros
    pending = start(0, 0)
    for c in range(NCH):
        slot = c & 1
        nxt = start(c + 1, slot ^ 1) if c + 1 < NCH else None
        for h in pending:
            h.wait()
        pending = nxt
        avs, bvs, mvs = abufs[slot], bbufs[slot], mbufs[slot]

        def inner(i, acc):
            s = i * L
            a = avs[pl.ds(s, L)]
            b = bvs[pl.ds(s, L)]
            m = mvs[pl.ds(s, L)]
            d = jnp.where(m != 0, jnp.abs(a - b), jnp.float32(-1.0))
            u = lax.bitcast_convert_type(d, jnp.int32)
            idx1 = lax.shift_right_logical(u, 20)
            if is_pass2:
                idx2 = lax.bitwise_and(lax.shift_right_logical(u, 9),
                                       jnp.int32(2047))
                idx = jnp.where(idx1 == bsel, idx2, jnp.int32(3000))
                acc = acc + jnp.where((idx1 > bsel) & (idx1 < 2048),
                                      d, jnp.float32(0.0))
            else:
                idx = idx1
            plsc.addupdate_scatter(hc, [idx], ones)
            return acc
        acc = lax.fori_loop(0, C // L, inner, acc, unroll=UNROLL)

    accv[...] = acc
    pltpu.sync_copy(hc, out_hbm.at[wid, pl.ds(0, NBINS)])
    pltpu.sync_copy(accv, out_hbm.at[wid, pl.ds(NBINS, L)])


def _make_pass(is_pass2):
    mesh = plsc.VectorSubcoreMesh(core_axis_name="c", subcore_axis_name="s",
                                  num_cores=NC, num_subcores=NS)
    return pl.kernel(
        functools.partial(_histo_body, is_pass2),
        out_type=jax.ShapeDtypeStruct((NW, NBINS + L), jnp.float32),
        mesh=mesh,
        scratch_types=[
            pltpu.VMEM((C,), jnp.float32),
            pltpu.VMEM((C,), jnp.float32),
            pltpu.VMEM((C,), jnp.float32),
            pltpu.VMEM((C,), jnp.float32),
            pltpu.VMEM((C,), jnp.int32),
            pltpu.VMEM((C,), jnp.int32),
            pltpu.VMEM((L,), jnp.int32),
            pltpu.VMEM((L,), jnp.float32),
            pltpu.VMEM((NBINS,), jnp.float32),
            pltpu.SemaphoreType.DMA((2,)),
            pltpu.SemaphoreType.DMA((2,)),
            pltpu.SemaphoreType.DMA((2,)),
        ],
        compiler_params=pltpu.CompilerParams(needs_layout_passes=False),
        name="ohem_histo2" if is_pass2 else "ohem_histo1",
    )


def _select_bin(cnt, k):
    cum = jnp.cumsum(cnt[::-1])[::-1]
    b = jnp.clip(jnp.sum((cum >= k).astype(jnp.int32)) - 1, 0, 2047)
    cump = jnp.concatenate([cum, jnp.zeros((1,), cum.dtype)])
    return b, cump[b + 1]


def kernel(inputs, targets, mask):
    a = inputs.reshape(-1)
    b = targets.reshape(-1)
    m = mask.reshape(-1).astype(jnp.int32)

    pass1 = _make_pass(False)
    pass2 = _make_pass(True)

    h1 = pass1(a, b, m, jnp.zeros((L,), jnp.int32))
    cnt1 = jnp.sum(h1[:, :2048], axis=0).astype(jnp.int32)
    num_selected = jnp.sum(cnt1)
    k = (num_selected * 6) // 10
    b1, cnt_gt1 = _select_bin(cnt1, k)

    h2 = pass2(a, b, m, jnp.full((L,), b1, jnp.int32))
    cnt2f = jnp.sum(h2[:, :2048], axis=0)
    cnt2 = cnt2f.astype(jnp.int32)
    sum_hi = jnp.sum(h2[:, NBINS:])

    k2 = k - cnt_gt1
    b2, cnt_gt2 = _select_bin(cnt2, k2)

    j = jnp.arange(2048, dtype=jnp.int32)
    mids = lax.bitcast_convert_type((b1 << 20) | (j << 9) | 256, jnp.float32)
    sum_mid = jnp.sum(jnp.where(j > b2, cnt2f * mids, 0.0))
    t_mid = lax.bitcast_convert_type((b1 << 20) | (b2 << 9) | 255, jnp.float32)
    rem = (k2 - cnt_gt2).astype(jnp.float32)
    total = sum_hi + sum_mid + rem * t_mid
    return total / k.astype(jnp.float32)


# speedup vs baseline: 18.1636x; 1.7293x over previous
import functools

import jax
import jax.numpy as jnp
from jax import lax
from jax.experimental import pallas as pl
from jax.experimental.pallas import tpu as pltpu
from jax.experimental.pallas import tpu_sc as plsc

NC = 2
NS = 16
NW = NC * NS
L = 16

N = 128 * 32768
E = N // NW
C = 8192
NCH = E // C
NBINS = 4096
UNROLL = 8


def _histo_body(is_pass2, a_hbm, b_hbm, m_hbm, bsel_hbm, out_hbm,
                av0, av1, bv0, bv1, mv0, mv1, bselv, accv, hc, sa, sb, sm):
    abufs, bbufs, mbufs = (av0, av1), (bv0, bv1), (mv0, mv1)
    wid = lax.axis_index("s") * NC + lax.axis_index("c")
    base = wid * E
    zeros = jnp.zeros((L,), jnp.float32)
    ones = jnp.ones((L,), jnp.float32)

    def zero_body(j, _):
        hc[pl.ds(j * L, L)] = zeros
        return _
    lax.fori_loop(0, NBINS // L, zero_body, None, unroll=8)

    if is_pass2:
        pltpu.sync_copy(bsel_hbm, bselv)
        bsel = bselv[...]

    def start(c, slot):
        off = base + c * C
        return (pltpu.async_copy(a_hbm.at[pl.ds(off, C)], abufs[slot], sa.at[slot]),
                pltpu.async_copy(b_hbm.at[pl.ds(off, C)], bbufs[slot], sb.at[slot]),
                pltpu.async_copy(m_hbm.at[pl.ds(off, C)], mbufs[slot], sm.at[slot]))

    acc = zeros
    pending = start(0, 0)
    for c in range(NCH):
        slot = c & 1
        nxt = start(c + 1, slot ^ 1) if c + 1 < NCH else None
        for h in pending:
            h.wait()
        pending = nxt
        avs, bvs, mvs = abufs[slot], bbufs[slot], mbufs[slot]

        def inner(i, acc):
            s = i * L
            a = avs[pl.ds(s, L)]
            b = bvs[pl.ds(s, L)]
            m = mvs[pl.ds(s, L)]
            sel = m != 0
            d = jnp.abs(a - b)
            u = lax.bitcast_convert_type(d, jnp.int32)
            idx1 = lax.shift_right_logical(u, 20)
            if is_pass2:
                idx = lax.bitwise_and(lax.shift_right_logical(u, 9),
                                      jnp.int32(2047))
                inb = sel & (idx1 == bsel)
                acc = acc + jnp.where(sel & (idx1 > bsel), d, jnp.float32(0.0))
            else:
                idx = idx1
                inb = sel
            plsc.addupdate_scatter(hc, [idx], ones, mask=inb)
            return acc
        acc = lax.fori_loop(0, C // L, inner, acc, unroll=UNROLL)

    accv[...] = acc
    pltpu.sync_copy(hc, out_hbm.at[wid, pl.ds(0, NBINS)])
    pltpu.sync_copy(accv, out_hbm.at[wid, pl.ds(NBINS, L)])


def _make_pass(is_pass2):
    mesh = plsc.VectorSubcoreMesh(core_axis_name="c", subcore_axis_name="s",
                                  num_cores=NC, num_subcores=NS)
    return pl.kernel(
        functools.partial(_histo_body, is_pass2),
        out_type=jax.ShapeDtypeStruct((NW, NBINS + L), jnp.float32),
        mesh=mesh,
        scratch_types=[
            pltpu.VMEM((C,), jnp.float32),
            pltpu.VMEM((C,), jnp.float32),
            pltpu.VMEM((C,), jnp.float32),
            pltpu.VMEM((C,), jnp.float32),
            pltpu.VMEM((C,), jnp.int32),
            pltpu.VMEM((C,), jnp.int32),
            pltpu.VMEM((L,), jnp.int32),
            pltpu.VMEM((L,), jnp.float32),
            pltpu.VMEM((NBINS,), jnp.float32),
            pltpu.SemaphoreType.DMA((2,)),
            pltpu.SemaphoreType.DMA((2,)),
            pltpu.SemaphoreType.DMA((2,)),
        ],
        compiler_params=pltpu.CompilerParams(needs_layout_passes=False),
        name="ohem_histo2" if is_pass2 else "ohem_histo1",
    )


def _select_bin(cnt, k):
    cum = jnp.cumsum(cnt[::-1])[::-1]
    b = jnp.clip(jnp.sum((cum >= k).astype(jnp.int32)) - 1, 0, 2047)
    cump = jnp.concatenate([cum, jnp.zeros((1,), cum.dtype)])
    return b, cump[b + 1]


def kernel(inputs, targets, mask):
    a = inputs.reshape(-1)
    b = targets.reshape(-1)
    m = mask.reshape(-1).astype(jnp.int32)

    pass1 = _make_pass(False)
    pass2 = _make_pass(True)

    h1 = pass1(a, b, m, jnp.zeros((L,), jnp.int32))
    cnt1 = jnp.sum(h1[:, :2048], axis=0).astype(jnp.int32)
    num_selected = jnp.sum(cnt1)
    k = (num_selected * 6) // 10
    b1, cnt_gt1 = _select_bin(cnt1, k)

    h2 = pass2(a, b, m, jnp.full((L,), b1, jnp.int32))
    cnt2f = jnp.sum(h2[:, :2048], axis=0)
    cnt2 = cnt2f.astype(jnp.int32)
    sum_hi = jnp.sum(h2[:, NBINS:])

    k2 = k - cnt_gt1
    b2, cnt_gt2 = _select_bin(cnt2, k2)

    j = jnp.arange(2048, dtype=jnp.int32)
    mids = lax.bitcast_convert_type((b1 << 20) | (j << 9) | 256, jnp.float32)
    sum_mid = jnp.sum(jnp.where(j > b2, cnt2f * mids, 0.0))
    t_mid = lax.bitcast_convert_type((b1 << 20) | (b2 << 9) | 255, jnp.float32)
    rem = (k2 - cnt_gt2).astype(jnp.float32)
    total = sum_hi + sum_mid + rem * t_mid
    return total / k.astype(jnp.float32)
